# baseline (speedup 1.0000x reference)
import functools
import math

import jax
import jax.numpy as jnp
import numpy as np
from jax import lax
from jax.experimental import pallas as pl
from jax.experimental.pallas import tpu as pltpu
from jax.experimental.pallas import tpu_sc as plsc

_D = 128
_LOG1E4 = math.log(10000.0)


def _pe_rows(indices):
    d = jnp.arange(_D)
    div = jnp.exp(((d // 2) * 2).astype(jnp.float32) * (-_LOG1E4 / _D))
    phase = indices.astype(jnp.float32)[:, None] * div[None, :]
    return jnp.where((d % 2 == 0)[None, :], jnp.sin(phase), jnp.cos(phase))


def _uv_const(block_b):
    d = np.arange(_D)
    div = np.exp(((d // 2) * 2).astype(np.float64) * (-_LOG1E4 / _D))
    ph = np.arange(block_b)[:, None] * div[None, :]
    even = (d % 2 == 0)[None, :]
    u = np.where(even, np.cos(ph), -np.sin(ph)).astype(np.float32)
    v = np.where(even, np.sin(ph), np.cos(ph)).astype(np.float32)
    return u, v


def _tab_body(raw_ref, lnnw_ref, lnnb_ref, w_ref, out_ref):
    raw = raw_ref[...]
    mu = jnp.mean(raw, axis=-1, keepdims=True)
    var = jnp.mean((raw - mu) ** 2, axis=-1, keepdims=True)
    lnc = (raw - mu) * jax.lax.rsqrt(var + 1e-12) * lnnw_ref[...] + lnnb_ref[...]
    ta = jax.lax.dot_general(lnc, w_ref[:, 128:256], (((1,), (1,)), ((), ())),
                             preferred_element_type=jnp.float32)
    tb = jax.lax.dot_general(lnc, w_ref[:, 256:384], (((1,), (1,)), ((), ())),
                             preferred_element_type=jnp.float32)
    pair = ta[:, None, :] + tb[None, :, :]
    out_ref[...] = pair.reshape(48 * 48, _D)


def _main_body(idx_ref, p1_ref, eop_ref, lnow_ref, lnob_ref, w_ref, bias_ref,
               u_ref, v_ref, out_ref, *, block_b):
    i = pl.program_id(0)

    dlane = jax.lax.broadcasted_iota(jnp.int32, (1, _D), 1)
    div = jnp.exp(((dlane // 2) * 2).astype(jnp.float32) * (-_LOG1E4 / _D))

    idx = idx_ref[...]
    opt = idx[0:1, :]
    sub16 = jax.lax.broadcasted_iota(jnp.int32, (16, block_b), 0)
    oh_opt = (opt == sub16).astype(jnp.float32)
    e_opt = jax.lax.dot_general(oh_opt, eop_ref[...], (((0,), (0,)), ((), ())),
                                preferred_element_type=jnp.float32)

    base = (i * block_b + 1).astype(jnp.float32) * div
    sb = jnp.sin(base)
    cb = jnp.cos(base)
    pe = sb * u_ref[...] + cb * v_ref[...]

    x = e_opt + pe
    mu = jnp.mean(x, axis=-1, keepdims=True)
    var = jnp.mean((x - mu) ** 2, axis=-1, keepdims=True)
    lx = (x - mu) * jax.lax.rsqrt(var + 1e-12) * lnow_ref[...] + lnob_ref[...]
    out = jax.lax.dot_general(lx, w_ref[:, 0:128], (((1,), (1,)), ((), ())),
                              preferred_element_type=jnp.float32)
    out_ref[...] = out + p1_ref[...] + bias_ref[...]


def _sc_gather(ptab, c01, n):
    info = plsc.get_sparse_core_info()
    nw = info.num_cores * info.num_subcores
    per_w = n // nw
    chunk = 512
    nchunk = per_w // chunk
    mesh = plsc.VectorSubcoreMesh(core_axis_name="c", subcore_axis_name="s")

    @functools.partial(
        pl.kernel, mesh=mesh,
        out_type=jax.ShapeDtypeStruct((n, _D), jnp.float32),
        scratch_types=[
            pltpu.VMEM((chunk,), jnp.int32),
            pltpu.VMEM((chunk, _D), jnp.float32),
            pltpu.SemaphoreType.DMA,
        ],
    )
    def k(ptab_hbm, idx_hbm, out_hbm, idx_v, rows_v, sem):
        wid = lax.axis_index("s") * info.num_cores + lax.axis_index("c")
        for c in range(nchunk):
            start = wid * per_w + c * chunk
            pltpu.sync_copy(idx_hbm.at[pl.ds(start, chunk)], idx_v)
            pltpu.async_copy(ptab_hbm.at[idx_v], rows_v, sem).wait()
            pltpu.sync_copy(rows_v, out_hbm.at[pl.ds(start, chunk)])

    return k(ptab, c01)


def kernel(opt_idx, opnd_type, opnd_idx, float_operand_emb, fixed_operator_emb,
           fixed_operand_emb, operand_table, operator_param, operand_param,
           ln_opt_w, ln_opt_b, ln_opnd_w, ln_opnd_b, W, b):
    n = opt_idx.shape[0]
    block_b = 2048
    nb = n // block_b

    pe16 = _pe_rows(jnp.arange(16) + 1)
    raw48 = (operand_param * jnp.repeat(operand_table, 16, axis=0)
             + jnp.concatenate([float_operand_emb[:16], pe16,
                                fixed_operand_emb[:16]], axis=0))
    eop = jnp.pad(operator_param * fixed_operator_emb,
                  ((0, 16 - fixed_operator_emb.shape[0]), (0, 0)))
    c0 = (opnd_type[:, 0] * 16 + opnd_idx[:, 0]).astype(jnp.int32)
    c1 = (opnd_type[:, 1] * 16 + opnd_idx[:, 1]).astype(jnp.int32)
    c01 = c0 * 48 + c1
    idx = jnp.stack([opt_idx.astype(jnp.int32), jnp.zeros_like(c0),
                     jnp.zeros_like(c0), jnp.zeros_like(c0)], axis=0)
    u_np, v_np = _uv_const(block_b)
    u_c = jnp.asarray(u_np)
    v_c = jnp.asarray(v_np)

    full = lambda a: pl.BlockSpec(a.shape, lambda i: (0,) * a.ndim)
    lnow = ln_opt_w.reshape(1, _D)
    lnob = ln_opt_b.reshape(1, _D)
    lnnw = ln_opnd_w.reshape(1, _D)
    lnnb = ln_opnd_b.reshape(1, _D)
    bias = b.reshape(1, _D)

    full0 = lambda a: pl.BlockSpec(a.shape, lambda: (0,) * a.ndim)
    ptab = pl.pallas_call(
        _tab_body,
        in_specs=[full0(raw48), full0(lnnw), full0(lnnb), full0(W)],
        out_specs=pl.BlockSpec((48 * 48, _D), lambda: (0, 0)),
        out_shape=jax.ShapeDtypeStruct((48 * 48, _D), jnp.float32),
    )(raw48, lnnw, lnnb, W)

    p1 = _sc_gather(ptab, c01, n)

    return pl.pallas_call(
        functools.partial(_main_body, block_b=block_b),
        grid=(nb,),
        in_specs=[
            pl.BlockSpec((4, block_b), lambda i: (0, i)),
            pl.BlockSpec((block_b, _D), lambda i: (i, 0)),
            full(eop), full(lnow), full(lnob), full(W), full(bias),
            full(u_c), full(v_c),
        ],
        out_specs=pl.BlockSpec((block_b, _D), lambda i: (i, 0)),
        out_shape=jax.ShapeDtypeStruct((n, _D), jnp.float32),
        compiler_params=pltpu.CompilerParams(
            dimension_semantics=("arbitrary",)),
    )(idx, p1, eop, lnow, lnob, W, bias, u_c, v_c)


# speedup vs baseline: 1.9065x; 1.9065x over previous
import functools
import math

import jax
import jax.numpy as jnp
import numpy as np
from jax.experimental import pallas as pl
from jax.experimental.pallas import tpu as pltpu

_D = 128
_LOG1E4 = math.log(10000.0)


def _uv_const(block_b):
    d = np.arange(_D)
    div = np.exp(((d // 2) * 2).astype(np.float64) * (-_LOG1E4 / _D))
    ph = np.arange(block_b)[:, None] * div[None, :]
    even = (d % 2 == 0)[None, :]
    u = np.where(even, np.cos(ph), -np.sin(ph)).astype(np.float32)
    v = np.where(even, np.sin(ph), np.cos(ph)).astype(np.float32)
    return u, v


def _pe_rows(indices):
    d = jnp.arange(_D)
    div = jnp.exp(((d // 2) * 2).astype(jnp.float32) * (-_LOG1E4 / _D))
    phase = indices.astype(jnp.float32)[:, None] * div[None, :]
    return jnp.where((d % 2 == 0)[None, :], jnp.sin(phase), jnp.cos(phase))


def _body(idx_ref, raw_ref, eop_ref, lnow_ref, lnob_ref,
          lnnw_ref, lnnb_ref, w_ref, bias_ref, u_ref, v_ref, out_ref, tab_ref,
          *, block_b):
    i = pl.program_id(0)

    dlane = jax.lax.broadcasted_iota(jnp.int32, (1, _D), 1)
    div = jnp.exp(((dlane // 2) * 2).astype(jnp.float32) * (-_LOG1E4 / _D))

    @pl.when(i == 0)
    def _build_tables():
        raw = raw_ref[...]
        mu = jnp.mean(raw, axis=-1, keepdims=True)
        var = jnp.mean((raw - mu) ** 2, axis=-1, keepdims=True)
        lnc = ((raw - mu) * jax.lax.rsqrt(var + 1e-12) * lnnw_ref[...]
               + lnnb_ref[...])
        tab_ref[0:48, :] = jax.lax.dot_general(
            lnc, w_ref[:, 128:256], (((1,), (1,)), ((), ())),
            preferred_element_type=jnp.float32)
        tab_ref[48:96, :] = jax.lax.dot_general(
            lnc, w_ref[:, 256:384], (((1,), (1,)), ((), ())),
            preferred_element_type=jnp.float32)

    idx = idx_ref[...]
    opt = idx[0:1, :]
    sub16 = jax.lax.broadcasted_iota(jnp.int32, (16, block_b), 0)
    oh_opt = (opt == sub16).astype(jnp.bfloat16)
    e_opt = jax.lax.dot_general(oh_opt, eop_ref[...].astype(jnp.bfloat16),
                                (((0,), (0,)), ((), ())),
                                preferred_element_type=jnp.float32)

    base = (i * block_b + 1).astype(jnp.float32) * div
    sb = jnp.sin(base)
    cb = jnp.cos(base)
    pe = sb * u_ref[...] + cb * v_ref[...]

    x = e_opt + pe
    mu = jnp.mean(x, axis=-1, keepdims=True)
    var = jnp.mean((x - mu) ** 2, axis=-1, keepdims=True)
    lx = (x - mu) * jax.lax.rsqrt(var + 1e-12) * lnow_ref[...] + lnob_ref[...]
    out = jax.lax.dot_general(lx, w_ref[:, 0:128], (((1,), (1,)), ((), ())),
                              preferred_element_type=jnp.float32)

    sub96 = jax.lax.broadcasted_iota(jnp.int32, (96, block_b), 0)
    oh2 = ((idx[1:2, :] == sub96) | (idx[2:3, :] == sub96)).astype(jnp.bfloat16)
    out = out + jax.lax.dot_general(oh2, tab_ref[...].astype(jnp.bfloat16),
                                    (((0,), (0,)), ((), ())),
                                    preferred_element_type=jnp.float32)
    out_ref[...] = out + bias_ref[...]


def kernel(opt_idx, opnd_type, opnd_idx, float_operand_emb, fixed_operator_emb,
           fixed_operand_emb, operand_table, operator_param, operand_param,
           ln_opt_w, ln_opt_b, ln_opnd_w, ln_opnd_b, W, b):
    n = opt_idx.shape[0]
    block_b = 2048
    nb = n // block_b

    pe16 = _pe_rows(jnp.arange(16) + 1)
    raw48 = (operand_param * jnp.repeat(operand_table, 16, axis=0)
             + jnp.concatenate([float_operand_emb[:16], pe16,
                                fixed_operand_emb[:16]], axis=0))
    eop = jnp.pad(operator_param * fixed_operator_emb,
                  ((0, 16 - fixed_operator_emb.shape[0]), (0, 0)))
    c0 = (opnd_type[:, 0] * 16 + opnd_idx[:, 0]).astype(jnp.int32)
    c1 = (opnd_type[:, 1] * 16 + opnd_idx[:, 1] + 48).astype(jnp.int32)
    idx = jnp.stack([opt_idx.astype(jnp.int32), c0, c1,
                     jnp.zeros_like(c0)], axis=0)
    u_np, v_np = _uv_const(block_b)
    u_c = jnp.asarray(u_np)
    v_c = jnp.asarray(v_np)

    full = lambda a: pl.BlockSpec(a.shape, lambda i: (0,) * a.ndim)
    lnow = ln_opt_w.reshape(1, _D)
    lnob = ln_opt_b.reshape(1, _D)
    lnnw = ln_opnd_w.reshape(1, _D)
    lnnb = ln_opnd_b.reshape(1, _D)
    bias = b.reshape(1, _D)

    return pl.pallas_call(
        functools.partial(_body, block_b=block_b),
        grid=(nb,),
        in_specs=[
            pl.BlockSpec((4, block_b), lambda i: (0, i)),
            full(raw48), full(eop), full(lnow), full(lnob), full(lnnw),
            full(lnnb), full(W), full(bias), full(u_c), full(v_c),
        ],
        out_specs=pl.BlockSpec((block_b, _D), lambda i: (i, 0)),
        out_shape=jax.ShapeDtypeStruct((n, _D), jnp.float32),
        scratch_shapes=[pltpu.VMEM((96, _D), jnp.float32)],
        compiler_params=pltpu.CompilerParams(
            dimension_semantics=("arbitrary",)),
    )(idx, raw48, eop, lnow, lnob, lnnw, lnnb, W, bias, u_c, v_c)


# speedup vs baseline: 1.9575x; 1.0268x over previous
import functools
import math

import jax
import jax.numpy as jnp
import numpy as np
from jax.experimental import pallas as pl
from jax.experimental.pallas import tpu as pltpu

_D = 128
_LOG1E4 = math.log(10000.0)


def _uv_const(block_b):
    d = np.arange(_D)
    div = np.exp(((d // 2) * 2).astype(np.float64) * (-_LOG1E4 / _D))
    ph = np.arange(block_b)[:, None] * div[None, :]
    even = (d % 2 == 0)[None, :]
    u = np.where(even, np.cos(ph), -np.sin(ph)).astype(np.float32)
    v = np.where(even, np.sin(ph), np.cos(ph)).astype(np.float32)
    return u, v


def _pe_rows(indices):
    d = jnp.arange(_D)
    div = jnp.exp(((d // 2) * 2).astype(jnp.float32) * (-_LOG1E4 / _D))
    phase = indices.astype(jnp.float32)[:, None] * div[None, :]
    return jnp.where((d % 2 == 0)[None, :], jnp.sin(phase), jnp.cos(phase))


def _body(idx_ref, raw_ref, eop_ref, lnow_ref, lnob_ref,
          lnnw_ref, lnnb_ref, w_ref, bias_ref, u_ref, v_ref, out_ref, tab_ref,
          *, block_b):
    i = pl.program_id(0)

    dlane = jax.lax.broadcasted_iota(jnp.int32, (1, _D), 1)
    div = jnp.exp(((dlane // 2) * 2).astype(jnp.float32) * (-_LOG1E4 / _D))

    @pl.when(i == 0)
    def _build_tables():
        raw = raw_ref[...]
        mu = jnp.mean(raw, axis=-1, keepdims=True)
        var = jnp.mean((raw - mu) ** 2, axis=-1, keepdims=True)
        lnc = ((raw - mu) * jax.lax.rsqrt(var + 1e-12) * lnnw_ref[...]
               + lnnb_ref[...])
        tab_ref[0:48, :] = jax.lax.dot_general(
            lnc, w_ref[:, 128:256], (((1,), (1,)), ((), ())),
            preferred_element_type=jnp.float32) + bias_ref[...]
        tab_ref[48:96, :] = jax.lax.dot_general(
            lnc, w_ref[:, 256:384], (((1,), (1,)), ((), ())),
            preferred_element_type=jnp.float32)

    idx = idx_ref[...]
    opt = idx[0:1, :]
    sub16 = jax.lax.broadcasted_iota(jnp.int32, (16, block_b), 0)
    oh_opt = (opt == sub16).astype(jnp.float32)
    e_opt = jax.lax.dot_general(oh_opt, eop_ref[...], (((0,), (0,)), ((), ())),
                                preferred_element_type=jnp.float32)

    base = (i * block_b + 1).astype(jnp.float32) * div
    sb = jnp.sin(base)
    cb = jnp.cos(base)
    pe = sb * u_ref[...] + cb * v_ref[...]

    x = e_opt + pe
    mu = jnp.mean(x, axis=-1, keepdims=True)
    var = jnp.mean((x - mu) ** 2, axis=-1, keepdims=True)
    lx = (x - mu) * jax.lax.rsqrt(var + 1e-12) * lnow_ref[...] + lnob_ref[...]
    out = jax.lax.dot_general(lx, w_ref[:, 0:128], (((1,), (1,)), ((), ())),
                              preferred_element_type=jnp.float32)

    sub96 = jax.lax.broadcasted_iota(jnp.int32, (96, block_b), 0)
    oh2 = ((idx[1:2, :] == sub96) | (idx[2:3, :] == sub96)).astype(jnp.float32)
    out = out + jax.lax.dot_general(oh2, tab_ref[...], (((0,), (0,)), ((), ())),
                                    preferred_element_type=jnp.float32)
    out_ref[...] = out


def kernel(opt_idx, opnd_type, opnd_idx, float_operand_emb, fixed_operator_emb,
           fixed_operand_emb, operand_table, operator_param, operand_param,
           ln_opt_w, ln_opt_b, ln_opnd_w, ln_opnd_b, W, b):
    n = opt_idx.shape[0]
    block_b = 2048
    nb = n // block_b

    pe16 = _pe_rows(jnp.arange(16) + 1)
    raw48 = (operand_param * jnp.repeat(operand_table, 16, axis=0)
             + jnp.concatenate([float_operand_emb[:16], pe16,
                                fixed_operand_emb[:16]], axis=0))
    eop = jnp.pad(operator_param * fixed_operator_emb,
                  ((0, 16 - fixed_operator_emb.shape[0]), (0, 0)))
    c0 = (opnd_type[:, 0] * 16 + opnd_idx[:, 0]).astype(jnp.int32)
    c1 = (opnd_type[:, 1] * 16 + opnd_idx[:, 1] + 48).astype(jnp.int32)
    idx = jnp.stack([opt_idx.astype(jnp.int32), c0, c1,
                     jnp.zeros_like(c0)], axis=0)
    u_np, v_np = _uv_const(block_b)
    u_c = jnp.asarray(u_np)
    v_c = jnp.asarray(v_np)

    full = lambda a: pl.BlockSpec(a.shape, lambda i: (0,) * a.ndim)
    lnow = ln_opt_w.reshape(1, _D)
    lnob = ln_opt_b.reshape(1, _D)
    lnnw = ln_opnd_w.reshape(1, _D)
    lnnb = ln_opnd_b.reshape(1, _D)
    bias = b.reshape(1, _D)

    return pl.pallas_call(
        functools.partial(_body, block_b=block_b),
        grid=(nb,),
        in_specs=[
            pl.BlockSpec((4, block_b), lambda i: (0, i)),
            full(raw48), full(eop), full(lnow), full(lnob), full(lnnw),
            full(lnnb), full(W), full(bias), full(u_c), full(v_c),
        ],
        out_specs=pl.BlockSpec((block_b, _D), lambda i: (i, 0)),
        out_shape=jax.ShapeDtypeStruct((n, _D), jnp.float32),
        scratch_shapes=[pltpu.VMEM((96, _D), jnp.float32)],
        compiler_params=pltpu.CompilerParams(
            dimension_semantics=("arbitrary",)),
    )(idx, raw48, eop, lnow, lnob, lnnw, lnnb, W, bias, u_c, v_c)
